# Initial kernel scaffold; baseline (speedup 1.0000x reference)
import jax
import jax.numpy as jnp
from jax.experimental import pallas as pl


def kernel(h, edge_vec, edge_scalars, edge_index, edge_type, params):
    raise NotImplementedError("write your pallas kernel here")


# speedup vs baseline: 1.6201x; 1.6201x over previous
import functools

import jax
import jax.numpy as jnp
import numpy as np
from jax.experimental import pallas as pl

N_SCALAR = 48
N_VEC = 16
FEAT = 96
N_CH = 64
EPS = 1e-8

_P_REP = np.kron(np.eye(N_VEC, dtype=np.float32), np.ones((1, 3), np.float32))
_T_REP = np.tile(np.eye(3, dtype=np.float32), (1, N_VEC))
_G_SUM = np.kron(np.eye(N_VEC, dtype=np.float32), np.ones((3, 1), np.float32))

_INV_S48 = 1.0 / np.sqrt(48.0)
_INV_S16 = 1.0 / np.sqrt(16.0)
_INV_S3 = 1.0 / np.sqrt(3.0)
_INV_S2 = 1.0 / np.sqrt(2.0)
_S3 = np.sqrt(3.0)


def _silu(x):
    return x * jax.nn.sigmoid(x)


def _expand3(m):
    return jnp.kron(m, jnp.eye(3, dtype=m.dtype))


def _edge_kernel(hs_ref, ev_ref, es_ref, et_ref,
                 wtr_ref, btr_ref, wrin_ref, brin_ref, wrout_ref, brout_ref,
                 wtp1_ref, wtp2_ref, wtp4_ref, w3x_ref,
                 wnm1_ref, bnm1_ref, wnm2_ref, bnm2_ref,
                 wg1_ref, bg1_ref, wg2_ref, bg2_ref, sigt_ref,
                 prep_ref, trep_ref, gsum_ref,
                 out_ref):
    f32 = jnp.float32
    hs = hs_ref[...]
    es = es_ref[...]
    ev = ev_ref[...]
    prep = prep_ref[...]
    trep = trep_ref[...]
    gsum = gsum_ref[...]

    dot = functools.partial(jnp.dot, preferred_element_type=f32)

    trunk = _silu(dot(es, wtr_ref[...]) + btr_ref[...])
    r_in = dot(trunk, wrin_ref[...]) + brin_ref[...]
    r_out = dot(trunk, wrout_ref[...]) + brout_ref[...]

    s_in = hs[:, :N_SCALAR] * (1.0 + r_in[:, :N_SCALAR])
    v_in = hs[:, N_SCALAR:] * dot(1.0 + r_in[:, N_SCALAR:], prep)

    d2 = jnp.sum(ev * ev, axis=1, keepdims=True) + EPS
    dd = jnp.sqrt(d2)
    y1 = (_S3 / dd) * ev
    y_rep = dot(y1, trep)

    o1 = dot(s_in, wtp1_ref[...]) * _INV_S48
    dots = dot(v_in * y_rep, gsum) * _INV_S3
    o4 = dot(dots, wtp4_ref[...]) * _INV_S16
    out_s = (o1 + o4) * _INV_S2

    t2 = dot(s_in, wtp2_ref[...])
    o2 = dot(t2, prep) * y_rep * _INV_S48
    o3 = dot(v_in, w3x_ref[...]) * _INV_S16
    out_v = (o2 + o3) * _INV_S2

    out_s = out_s * (1.0 + r_out[:, :N_SCALAR])
    out_v = out_v * dot(1.0 + r_out[:, N_SCALAR:], prep)

    ms = _silu(out_s)
    norms = jnp.sqrt(dot(out_v * out_v, gsum) + EPS)
    gh = _silu(dot(norms, wnm1_ref[...]) + bnm1_ref[...])
    vg_act = jax.nn.sigmoid(dot(gh, wnm2_ref[...]) + bnm2_ref[...])
    mv = out_v * dot(vg_act, prep)

    ghid = _silu(dot(es, wg1_ref[...]) + bg1_ref[...])
    gates = dot(ghid, wg2_ref[...]) + bg2_ref[...]

    et = et_ref[...]
    lanes = jax.lax.broadcasted_iota(jnp.int32, (et.shape[0], 16), 1)
    onehot = (et == lanes).astype(f32)
    sigma = dot(onehot, sigt_ref[...])
    decay = jnp.exp(-dd / (sigma + EPS))
    s_gate = jax.nn.sigmoid(gates[:, 0:1]) * decay
    v_gate = jax.nn.sigmoid(gates[:, 1:17]) * decay

    num_s = s_gate * ms
    mv_c = dot(v_gate, prep) * mv
    b = num_s.shape[0]
    out_ref[0] = num_s[:, :32]
    out_ref[1] = jnp.concatenate([num_s[:, 32:48], mv_c[:, :16]], axis=1)
    out_ref[2] = mv_c[:, 16:48]
    out_ref[3] = jnp.concatenate(
        [v_gate, s_gate, jnp.zeros((b, 15), f32)], axis=1)


def _epilogue_kernel(acc_ref, h_ref,
                     wresc_ref, bresc_ref, wss_ref, bss_ref, wsvx_ref,
                     prep_ref, gsum_ref,
                     out_ref):
    f32 = jnp.float32
    dot = functools.partial(jnp.dot, preferred_element_type=f32)
    prep = prep_ref[...]
    gsum = gsum_ref[...]
    h = h_ref[...]

    accf = jnp.concatenate([acc_ref[0], acc_ref[1], acc_ref[2], acc_ref[3]],
                           axis=1)
    num_s = accf[:, :N_SCALAR]
    mv_sum = accf[:, N_SCALAR:96]
    den_v = accf[:, 96:112]
    den_s = accf[:, 112:113]

    agg_s = num_s / (den_s + EPS)
    agg_v = mv_sum / (dot(den_v, prep) + EPS)
    mag = jnp.sqrt(dot(agg_v * agg_v, gsum) + EPS)
    new_mag = mag * jax.nn.sigmoid(dot(mag, wresc_ref[...]) + bresc_ref[...])
    agg_v = agg_v * dot(new_mag / mag, prep)

    self_s = dot(h[:, :N_SCALAR], wss_ref[...]) + bss_ref[...]
    self_v = dot(h[:, N_SCALAR:], wsvx_ref[...])
    out_ref[...] = jnp.concatenate([agg_s + self_s, agg_v + self_v], axis=1)


def _pick_block(n, candidates):
    for c in candidates:
        if n % c == 0:
            return c
    return n


def _full_spec(shape):
    return pl.BlockSpec(shape, lambda i: tuple(0 for _ in shape))


def kernel(h, edge_vec, edge_scalars, edge_index, edge_type, params):
    p = params
    n, _ = h.shape
    e = edge_vec.shape[0]
    src = edge_index[0]
    dst = edge_index[1]

    prep = jnp.asarray(_P_REP)
    trep = jnp.asarray(_T_REP)
    gsum = jnp.asarray(_G_SUM)

    w3x = _expand3(p['W_tp3'])
    wsvx = _expand3(p['W_self_v'].T)
    sig_t = jnp.zeros((16, 1), jnp.float32).at[:10, 0].set(
        jnp.exp(p['log_sigma'][:, 0]))

    weights = [
        p['W_tr'].T, p['b_tr'][None, :],
        p['W_rin'].T, p['b_rin'][None, :],
        p['W_rout'].T, p['b_rout'][None, :],
        p['W_tp1'], p['W_tp2'], p['W_tp4'], w3x,
        p['W_nm1'].T, p['b_nm1'][None, :],
        p['W_nm2'].T, p['b_nm2'][None, :],
        p['W_g1'].T, p['b_g1'][None, :],
        p['W_g2'].T, p['b_g2'][None, :], sig_t,
        prep, trep, gsum,
    ]

    h_src = jnp.take(h, src, axis=0)

    be = _pick_block(e, [4000, 2000, 1000, 500, 200, 100, 50, 10])
    et2 = edge_type[:, None]
    edge_in_specs = [
        pl.BlockSpec((be, FEAT), lambda i: (i, 0)),
        pl.BlockSpec((be, 3), lambda i: (i, 0)),
        pl.BlockSpec((be, 16), lambda i: (i, 0)),
        pl.BlockSpec((be, 1), lambda i: (i, 0)),
    ] + [_full_spec(w.shape) for w in weights]

    payload = pl.pallas_call(
        _edge_kernel,
        grid=(e // be,),
        in_specs=edge_in_specs,
        out_specs=pl.BlockSpec((4, be, 32), lambda i: (0, i, 0)),
        out_shape=jax.ShapeDtypeStruct((4, e, 32), jnp.float32),
    )(h_src, edge_vec, edge_scalars, et2, *weights)

    flat = payload.transpose(1, 0, 2).reshape(e, 128)
    acc = jax.ops.segment_sum(flat, dst, num_segments=n)
    acc4 = acc.reshape(n, 4, 32).transpose(1, 0, 2)

    ep_weights = [
        p['W_resc'].T, p['b_resc'][None, :],
        p['W_self_s'].T, p['b_self_s'][None, :], wsvx,
        prep, gsum,
    ]
    bn = _pick_block(n, [2000, 1000, 500, 200, 100, 50, 10])
    out = pl.pallas_call(
        _epilogue_kernel,
        grid=(n // bn,),
        in_specs=[
            pl.BlockSpec((4, bn, 32), lambda i: (0, i, 0)),
            pl.BlockSpec((bn, FEAT), lambda i: (i, 0)),
        ] + [_full_spec(w.shape) for w in ep_weights],
        out_specs=pl.BlockSpec((bn, FEAT), lambda i: (i, 0)),
        out_shape=jax.ShapeDtypeStruct((n, FEAT), jnp.float32),
    )(acc4, h, *ep_weights)
    return out
